# Initial kernel scaffold; baseline (speedup 1.0000x reference)
import jax
import jax.numpy as jnp
from jax.experimental import pallas as pl


def kernel(input, index):
    raise NotImplementedError("write your pallas kernel here")


# speedup vs baseline: 2.2166x; 2.2166x over previous
import functools

import jax
import jax.numpy as jnp
from jax import lax
from jax.experimental import pallas as pl
from jax.experimental.pallas import tpu as pltpu
from jax.experimental.pallas import tpu_sc as plsc

D = 128
NCOL = D + 16
NC = 2
NS = 16
NW = NC * NS
S_PAD = 10240
CH = 80
CH3 = 200


def _k1_body(in_hbm, idx_hbm, zeros_hbm, part_hbm,
             rows0, rows1, idxb0, idxb1,
             stg_s0, stg_c0, pix_s0, pix_c0,
             stg_s1, stg_c1, pix_s1, pix_c1,
             spmem,
             sem0, sem1, fsem0, fsem1):
    c = lax.axis_index("c")
    s = lax.axis_index("s")
    wid = c * NS + s
    n = idx_hbm.shape[0]
    rpt = n // NW
    nch = rpt // CH
    base = wid * rpt
    tab_elems = S_PAD * NCOL // NS
    lanes = lax.iota(jnp.int32, 16)

    pltpu.sync_copy(zeros_hbm.at[pl.ds(s * tab_elems, tab_elems)],
                    spmem.at[pl.ds(s * tab_elems, tab_elems)])
    plsc.subcore_barrier()

    def start(rows_b, idx_b, ch, sem):
        pltpu.async_copy(in_hbm.at[pl.ds((base + ch * CH) * D, CH * D)],
                         rows_b, sem)
        pltpu.async_copy(idx_hbm.at[pl.ds(base + ch * CH, CH)],
                         idx_b.at[pl.ds(0, CH)], sem)

    def wait(rows_b, idx_b, ch, sem):
        pltpu.make_async_copy(
            in_hbm.at[pl.ds((base + ch * CH) * D, CH * D)], rows_b, sem).wait()
        pltpu.make_async_copy(
            idx_hbm.at[pl.ds(base + ch * CH, CH)],
            idx_b.at[pl.ds(0, CH)], sem).wait()

    sets = ((stg_s0, stg_c0, pix_s0, pix_c0, fsem0),
            (stg_s1, stg_c1, pix_s1, pix_c1, fsem1))

    def fwait(st):
        stg_s, stg_c, pix_s, pix_c, fsem = st
        pltpu.make_async_copy(stg_s, spmem.at[pix_s], fsem).wait()
        pltpu.make_async_copy(stg_c, spmem.at[pix_c], fsem).wait()

    def ffire(st, seg, cnt, accs):
        stg_s, stg_c, pix_s, pix_c, fsem = st
        ebase = seg * NCOL
        for j in range(D // 16):
            stg_s[pl.ds(16 * j, 16)] = accs[j]
            pix_s[pl.ds(16 * j, 16)] = ebase + 16 * j + lanes
        stg_c[...] = jnp.full((16,), cnt, jnp.float32)
        pix_c[...] = ebase + D + lanes
        pltpu.async_copy(stg_s, spmem.at[pix_s], fsem, add=True)
        pltpu.async_copy(stg_c, spmem.at[pix_c], fsem, add=True)

    def flush(seg, cnt, accs, k, do_flush):
        par = lax.rem(k, 2)
        for p in range(2):
            @pl.when(jnp.logical_and(do_flush,
                                     jnp.logical_and(par == p, k >= 2)))
            def _():
                fwait(sets[p])

            @pl.when(jnp.logical_and(do_flush, par == p))
            def _():
                ffire(sets[p], seg, cnt, accs)

        return jnp.where(do_flush, k + 1, k)

    def process(rows_b, idx_b, carry):
        def row_body(r, carry):
            seg, cnt, k = carry[0], carry[1], carry[2]
            accs = carry[3:]
            sv = idx_b[pl.ds(r, 16)][0]
            same = sv == seg
            do_flush = jnp.logical_and(jnp.logical_not(same), cnt > 0.0)
            k = flush(seg, cnt, accs, k, do_flush)
            new = []
            for j in range(D // 16):
                rj = rows_b[pl.ds(r * D + 16 * j, 16)]
                new.append(jnp.where(same, accs[j] + rj, rj))
            ncnt = jnp.where(same, cnt + 1.0, 1.0)
            return (sv, ncnt, k) + tuple(new)

        return lax.fori_loop(0, CH, row_body, carry)

    init = (jnp.int32(-1), jnp.float32(0.0), jnp.int32(0)) + tuple(
        jnp.zeros((16,), jnp.float32) for _ in range(D // 16))

    start(rows0, idxb0, 0, sem0)
    start(rows1, idxb1, 1, sem1)

    def pair_body(p, carry):
        ch0 = 2 * p
        wait(rows0, idxb0, ch0, sem0)
        carry = process(rows0, idxb0, carry)

        @pl.when(ch0 + 2 < nch)
        def _():
            start(rows0, idxb0, ch0 + 2, sem0)

        ch1 = 2 * p + 1
        wait(rows1, idxb1, ch1, sem1)
        carry = process(rows1, idxb1, carry)

        @pl.when(ch1 + 2 < nch)
        def _():
            start(rows1, idxb1, ch1 + 2, sem1)

        return carry

    carry = lax.fori_loop(0, nch // 2, pair_body, init)
    if nch % 2:
        wait(rows0, idxb0, nch - 1, sem0)
        carry = process(rows0, idxb0, carry)

    seg, cnt, k = carry[0], carry[1], carry[2]
    accs = carry[3:]
    k = flush(seg, cnt, accs, k, cnt > 0.0)
    for p in range(2):
        @pl.when(k >= p + 1)
        def _():
            fwait(sets[p])

    plsc.subcore_barrier()
    pltpu.sync_copy(spmem.at[pl.ds(s * tab_elems, tab_elems)],
                    part_hbm.at[c, pl.ds(s * tab_elems, tab_elems)])


def _k2_body(p_ref, o_ref):
    p0 = p_ref[0]
    p1 = p_ref[1]
    tot = p0[:, :D] + p1[:, :D]
    cnt = jnp.maximum(p0[:, D:D + 1] + p1[:, D:D + 1], 1.0)
    o_ref[...] = tot / cnt


def _k3_body(means_hbm, idx_hbm, out_hbm,
             idxb0, idxb1, rb0, rb1, si0, si1, sg, so0, so1):
    c = lax.axis_index("c")
    s = lax.axis_index("s")
    wid = c * NS + s
    n = out_hbm.shape[0]
    rpt = n // NW
    nch = rpt // CH3
    base = wid * rpt

    def istart(idx_b, ch, sem):
        pltpu.async_copy(idx_hbm.at[pl.ds(base + ch * CH3, CH3)], idx_b, sem)

    def iwait(idx_b, ch, sem):
        pltpu.make_async_copy(
            idx_hbm.at[pl.ds(base + ch * CH3, CH3)], idx_b, sem).wait()

    def owait(rows_b, ch, sem):
        pltpu.make_async_copy(
            rows_b, out_hbm.at[pl.ds(base + ch * CH3, CH3)], sem).wait()

    istart(idxb0, 0, si0)
    istart(idxb1, 1, si1)

    def pair_body(p, carry):
        ch0 = 2 * p
        iwait(idxb0, ch0, si0)

        @pl.when(p > 0)
        def _():
            owait(rb0, ch0 - 2, so0)

        pltpu.async_copy(means_hbm.at[idxb0], rb0, sg).wait()

        @pl.when(ch0 + 2 < nch)
        def _():
            istart(idxb0, ch0 + 2, si0)

        pltpu.async_copy(rb0, out_hbm.at[pl.ds(base + ch0 * CH3, CH3)], so0)

        ch1 = 2 * p + 1
        iwait(idxb1, ch1, si1)

        @pl.when(p > 0)
        def _():
            owait(rb1, ch1 - 2, so1)

        pltpu.async_copy(means_hbm.at[idxb1], rb1, sg).wait()

        @pl.when(ch1 + 2 < nch)
        def _():
            istart(idxb1, ch1 + 2, si1)

        pltpu.async_copy(rb1, out_hbm.at[pl.ds(base + ch1 * CH3, CH3)], so1)
        return carry

    lax.fori_loop(0, nch // 2, pair_body, 0)
    owait(rb0, nch - 2, so0)
    owait(rb1, nch - 1, so1)


def kernel(input, index):
    n, d = input.shape
    assert d == D
    idx32 = index.astype(jnp.int32)
    flat = input.reshape(-1)
    zeros = jnp.zeros((S_PAD * NCOL,), jnp.float32)

    mesh = plsc.VectorSubcoreMesh(core_axis_name="c", subcore_axis_name="s")

    k1 = functools.partial(
        pl.kernel,
        mesh=mesh,
        out_type=jax.ShapeDtypeStruct((NC, S_PAD * NCOL), jnp.float32),
        scratch_types=[
            pltpu.VMEM((CH * D,), jnp.float32),
            pltpu.VMEM((CH * D,), jnp.float32),
            pltpu.VMEM((CH + 16,), jnp.int32),
            pltpu.VMEM((CH + 16,), jnp.int32),
            pltpu.VMEM((D,), jnp.float32),
            pltpu.VMEM((16,), jnp.float32),
            pltpu.VMEM((D,), jnp.int32),
            pltpu.VMEM((16,), jnp.int32),
            pltpu.VMEM((D,), jnp.float32),
            pltpu.VMEM((16,), jnp.float32),
            pltpu.VMEM((D,), jnp.int32),
            pltpu.VMEM((16,), jnp.int32),
            pltpu.VMEM_SHARED((S_PAD * NCOL,), jnp.float32),
            pltpu.SemaphoreType.DMA,
            pltpu.SemaphoreType.DMA,
            pltpu.SemaphoreType.DMA,
            pltpu.SemaphoreType.DMA,
        ],
    )(_k1_body)
    parts = k1(flat, idx32, zeros)

    bs = 1024
    means = pl.pallas_call(
        _k2_body,
        grid=(S_PAD // bs,),
        in_specs=[pl.BlockSpec((NC, bs, NCOL), lambda i: (0, i, 0))],
        out_specs=pl.BlockSpec((bs, D), lambda i: (i, 0)),
        out_shape=jax.ShapeDtypeStruct((S_PAD, D), jnp.float32),
    )(parts.reshape(NC, S_PAD, NCOL))

    k3 = functools.partial(
        pl.kernel,
        mesh=mesh,
        out_type=jax.ShapeDtypeStruct((n, D), jnp.float32),
        scratch_types=[
            pltpu.VMEM((CH3,), jnp.int32),
            pltpu.VMEM((CH3,), jnp.int32),
            pltpu.VMEM((CH3, D), jnp.float32),
            pltpu.VMEM((CH3, D), jnp.float32),
            pltpu.SemaphoreType.DMA,
            pltpu.SemaphoreType.DMA,
            pltpu.SemaphoreType.DMA,
            pltpu.SemaphoreType.DMA,
            pltpu.SemaphoreType.DMA,
        ],
    )(_k3_body)
    out = k3(means, idx32)
    return out


# speedup vs baseline: 2.4940x; 1.1251x over previous
import functools

import jax
import jax.numpy as jnp
from jax import lax
from jax.experimental import pallas as pl
from jax.experimental.pallas import tpu as pltpu
from jax.experimental.pallas import tpu_sc as plsc

D = 128
NCOL = D + 16
NC = 2
NS = 16
NW = NC * NS
S_PAD = 10240
CH = 80
CH3 = 400


def _k1_body(in_hbm, idx_hbm, zeros_hbm, part_hbm,
             rows0, rows1, idxb0, idxb1, abuf0,
             stg_s0, stg_c0, pix_s0, pix_c0,
             stg_s1, stg_c1, pix_s1, pix_c1,
             spmem,
             sem0, sem1, fsem0, fsem1):
    c = lax.axis_index("c")
    s = lax.axis_index("s")
    wid = c * NS + s
    n = idx_hbm.shape[0]
    rpt = n // NW
    nch = rpt // CH
    base = wid * rpt
    tab_elems = S_PAD * NCOL // NS
    lanes = lax.iota(jnp.int32, 16)

    pltpu.sync_copy(zeros_hbm.at[pl.ds(s * tab_elems, tab_elems)],
                    spmem.at[pl.ds(s * tab_elems, tab_elems)])
    plsc.subcore_barrier()

    def start(rows_b, idx_b, ch, sem):
        pltpu.async_copy(in_hbm.at[pl.ds((base + ch * CH) * D, CH * D)],
                         rows_b, sem)
        pltpu.async_copy(idx_hbm.at[pl.ds(base + ch * CH, CH)],
                         idx_b.at[pl.ds(0, CH)], sem)

    def wait(rows_b, idx_b, ch, sem):
        pltpu.make_async_copy(
            in_hbm.at[pl.ds((base + ch * CH) * D, CH * D)], rows_b, sem).wait()
        pltpu.make_async_copy(
            idx_hbm.at[pl.ds(base + ch * CH, CH)],
            idx_b.at[pl.ds(0, CH)], sem).wait()

    sets = ((stg_s0, stg_c0, pix_s0, pix_c0, fsem0),
            (stg_s1, stg_c1, pix_s1, pix_c1, fsem1))

    def fwait(st):
        stg_s, stg_c, pix_s, pix_c, fsem = st
        pltpu.make_async_copy(stg_s, spmem.at[pix_s], fsem).wait()
        pltpu.make_async_copy(stg_c, spmem.at[pix_c], fsem).wait()

    def ffire(st, seg, cnt, abuf):
        stg_s, stg_c, pix_s, pix_c, fsem = st
        ebase = seg * NCOL
        for j in range(D // 16):
            stg_s[pl.ds(16 * j, 16)] = abuf[pl.ds(16 * j, 16)]
            pix_s[pl.ds(16 * j, 16)] = ebase + 16 * j + lanes
        stg_c[...] = jnp.full((16,), cnt, jnp.float32)
        pix_c[...] = ebase + D + lanes
        pltpu.async_copy(stg_s, spmem.at[pix_s], fsem, add=True)
        pltpu.async_copy(stg_c, spmem.at[pix_c], fsem, add=True)

    def flush(seg, cnt, abuf, k, do_flush):
        par = lax.rem(k, 2)
        for p in range(2):
            @pl.when(jnp.logical_and(do_flush,
                                     jnp.logical_and(par == p, k >= 2)))
            def _():
                fwait(sets[p])

            @pl.when(jnp.logical_and(do_flush, par == p))
            def _():
                ffire(sets[p], seg, cnt, abuf)

        return jnp.where(do_flush, k + 1, k)

    def process(rows_b, idx_b, abuf, carry):
        def row_range(lo, cnt16, carry):
            def row_body(r, carry):
                seg, cnt, k = carry
                sv = idx_b[pl.ds(r, 16)][0]
                same = sv == seg
                do_flush = jnp.logical_and(jnp.logical_not(same), cnt > 0.0)
                k = flush(seg, cnt, abuf, k, do_flush)
                for j in range(D // 16):
                    rj = rows_b[pl.ds(r * D + 16 * j, 16)]
                    aj = abuf[pl.ds(16 * j, 16)]
                    abuf[pl.ds(16 * j, 16)] = jnp.where(same, aj + rj, rj)
                ncnt = jnp.where(same, cnt + 1.0, 1.0)
                return (sv, ncnt, k)

            return lax.fori_loop(lo, lo + cnt16, row_body, carry)

        def group_body(g, carry):
            gb = g * 16
            v = idx_b[pl.ds(gb, 16)]
            seg = carry[0]
            uniform = jnp.logical_and(v[0] == seg, v[15] == seg)

            def fast_fn(carry):
                seg, cnt, k = carry
                for j in range(D // 16):
                    parts = [rows_b[pl.ds((gb + r) * D + 16 * j, 16)]
                             for r in range(16)]
                    while len(parts) > 1:
                        parts = [parts[i] + parts[i + 1]
                                 for i in range(0, len(parts), 2)]
                    abuf[pl.ds(16 * j, 16)] = abuf[pl.ds(16 * j, 16)] + parts[0]
                return (seg, cnt + 16.0, k)

            def slow_fn(carry):
                return row_range(gb, 16, carry)

            return lax.cond(uniform, fast_fn, slow_fn, carry)

        return lax.fori_loop(0, CH // 16, group_body, carry)

    init = (jnp.int32(-1), jnp.float32(0.0), jnp.int32(0))

    start(rows0, idxb0, 0, sem0)
    start(rows1, idxb1, 1, sem1)

    zv16 = jnp.zeros((16,), jnp.float32)
    for j in range(D // 16):
        abuf0[pl.ds(16 * j, 16)] = zv16

    def pair_body(p, carry):
        ch0 = 2 * p
        wait(rows0, idxb0, ch0, sem0)
        carry = process(rows0, idxb0, abuf0, carry)

        @pl.when(ch0 + 2 < nch)
        def _():
            start(rows0, idxb0, ch0 + 2, sem0)

        ch1 = 2 * p + 1
        wait(rows1, idxb1, ch1, sem1)
        carry = process(rows1, idxb1, abuf0, carry)

        @pl.when(ch1 + 2 < nch)
        def _():
            start(rows1, idxb1, ch1 + 2, sem1)

        return carry

    carry = lax.fori_loop(0, nch // 2, pair_body, init)
    if nch % 2:
        wait(rows0, idxb0, nch - 1, sem0)
        carry = process(rows0, idxb0, abuf0, carry)

    seg, cnt, k = carry[0], carry[1], carry[2]
    k = flush(seg, cnt, abuf0, k, cnt > 0.0)
    for p in range(2):
        @pl.when(k >= p + 1)
        def _():
            fwait(sets[p])

    plsc.subcore_barrier()
    pltpu.sync_copy(spmem.at[pl.ds(s * tab_elems, tab_elems)],
                    part_hbm.at[c, pl.ds(s * tab_elems, tab_elems)])


def _k2_body(p_ref, o_ref):
    p0 = p_ref[0]
    p1 = p_ref[1]
    tot = p0[:, :D] + p1[:, :D]
    cnt = jnp.maximum(p0[:, D:D + 1] + p1[:, D:D + 1], 1.0)
    o_ref[...] = tot / cnt


def _k3_body(means_hbm, idx_hbm, out_hbm,
             iq0, iq1, iq2, iq3, rb0, rb1,
             si0, si1, si2, si3, sg0, sg1, so0, so1):
    c = lax.axis_index("c")
    s = lax.axis_index("s")
    wid = c * NS + s
    n = out_hbm.shape[0]
    rpt = n // NW
    nch = rpt // CH3
    base = wid * rpt
    iq = (iq0, iq1, iq2, iq3)
    si = (si0, si1, si2, si3)
    rb = (rb0, rb1)
    sg = (sg0, sg1)
    so = (so0, so1)

    def istart(ch, q):
        pltpu.async_copy(idx_hbm.at[pl.ds(base + ch * CH3, CH3)], iq[q], si[q])

    def iwait(ch, q):
        pltpu.make_async_copy(
            idx_hbm.at[pl.ds(base + ch * CH3, CH3)], iq[q], si[q]).wait()

    def gstart(p, q):
        pltpu.async_copy(means_hbm.at[iq[q]], rb[p], sg[p])

    def gwait(p, q):
        pltpu.make_async_copy(means_hbm.at[iq[q]], rb[p], sg[p]).wait()

    def ostart(ch, p):
        pltpu.async_copy(rb[p], out_hbm.at[pl.ds(base + ch * CH3, CH3)], so[p])

    def owait(ch, p):
        pltpu.make_async_copy(
            rb[p], out_hbm.at[pl.ds(base + ch * CH3, CH3)], so[p]).wait()

    istart(0, 0)
    istart(1, 1)

    def body(i, carry):
        for p in range(2):
            is_p = lax.rem(i, 2) == p

            @pl.when(jnp.logical_and(is_p, i >= 1))
            def _():
                for q in range(4):
                    @pl.when(lax.rem(i - 1, 4) == q)
                    def _():
                        gwait(1 - p, q)
                ostart(i - 1, 1 - p)

            @pl.when(jnp.logical_and(is_p, i >= 2))
            def _():
                owait(i - 2, p)

            for q in range(4):
                is_q = jnp.logical_and(is_p, lax.rem(i, 4) == q)

                @pl.when(is_q)
                def _():
                    iwait(i, q)
                    gstart(p, q)

                @pl.when(jnp.logical_and(is_q, i + 2 < nch))
                def _():
                    istart(i + 2, (q + 2) % 4)
        return carry

    lax.fori_loop(0, nch, body, 0)

    lastp = (nch - 1) % 2
    gwait(lastp, (nch - 1) % 4)
    ostart(nch - 1, lastp)
    if nch >= 2:
        owait(nch - 2, 1 - lastp)
    owait(nch - 1, lastp)


def kernel(input, index):
    n, d = input.shape
    assert d == D
    idx32 = index.astype(jnp.int32)
    flat = input.reshape(-1)
    zeros = jnp.zeros((S_PAD * NCOL,), jnp.float32)

    mesh = plsc.VectorSubcoreMesh(core_axis_name="c", subcore_axis_name="s")

    k1 = functools.partial(
        pl.kernel,
        mesh=mesh,
        out_type=jax.ShapeDtypeStruct((NC, S_PAD * NCOL), jnp.float32),
        scratch_types=[
            pltpu.VMEM((CH * D,), jnp.float32),
            pltpu.VMEM((CH * D,), jnp.float32),
            pltpu.VMEM((CH + 16,), jnp.int32),
            pltpu.VMEM((CH + 16,), jnp.int32),
            pltpu.VMEM((D,), jnp.float32),
            pltpu.VMEM((D,), jnp.float32),
            pltpu.VMEM((16,), jnp.float32),
            pltpu.VMEM((D,), jnp.int32),
            pltpu.VMEM((16,), jnp.int32),
            pltpu.VMEM((D,), jnp.float32),
            pltpu.VMEM((16,), jnp.float32),
            pltpu.VMEM((D,), jnp.int32),
            pltpu.VMEM((16,), jnp.int32),
            pltpu.VMEM_SHARED((S_PAD * NCOL,), jnp.float32),
            pltpu.SemaphoreType.DMA,
            pltpu.SemaphoreType.DMA,
            pltpu.SemaphoreType.DMA,
            pltpu.SemaphoreType.DMA,
        ],
    )(_k1_body)
    parts = k1(flat, idx32, zeros)

    bs = 1024
    means = pl.pallas_call(
        _k2_body,
        grid=(S_PAD // bs,),
        in_specs=[pl.BlockSpec((NC, bs, NCOL), lambda i: (0, i, 0))],
        out_specs=pl.BlockSpec((bs, D), lambda i: (i, 0)),
        out_shape=jax.ShapeDtypeStruct((S_PAD, D), jnp.float32),
    )(parts.reshape(NC, S_PAD, NCOL))

    k3 = functools.partial(
        pl.kernel,
        mesh=mesh,
        out_type=jax.ShapeDtypeStruct((n, D), jnp.float32),
        scratch_types=[
            pltpu.VMEM((CH3,), jnp.int32),
            pltpu.VMEM((CH3,), jnp.int32),
            pltpu.VMEM((CH3,), jnp.int32),
            pltpu.VMEM((CH3,), jnp.int32),
            pltpu.VMEM((CH3, D), jnp.float32),
            pltpu.VMEM((CH3, D), jnp.float32),
            pltpu.SemaphoreType.DMA,
            pltpu.SemaphoreType.DMA,
            pltpu.SemaphoreType.DMA,
            pltpu.SemaphoreType.DMA,
            pltpu.SemaphoreType.DMA,
            pltpu.SemaphoreType.DMA,
            pltpu.SemaphoreType.DMA,
            pltpu.SemaphoreType.DMA,
        ],
    )(_k3_body)
    out = k3(means, idx32)
    return out
